# Initial kernel scaffold; baseline (speedup 1.0000x reference)
import jax
import jax.numpy as jnp
from jax.experimental import pallas as pl


def kernel(x, linear_w, linear_b, feature_mask, bn1_scale, bn1_bias, bn_a_scale, bn_a_bias, bn_b_scale, bn_b_bias, wt, conv_b, idx_in, idx_out):
    raise NotImplementedError("write your pallas kernel here")


# speedup vs baseline: 5.4556x; 5.4556x over previous
import math
from functools import partial

import jax
import jax.numpy as jnp
from jax.experimental import pallas as pl
from jax.experimental.pallas import tpu as pltpu

V_J = 25
_BITS = (1, 2, 4, 8, 16)
_VMEM_LIMIT = 64 * 1024 * 1024


def _fused_kernel(x_ref, m_ref, s1_ref, b1_ref, w_ref, wt_ref, bt_ref,
                  sb_ref, bb_ref, o_ref, *, tm):
    c = x_ref.shape[-1]
    xb = jnp.concatenate([x_ref[v] for v in range(V_J)], axis=0)
    s_lane = jax.lax.broadcasted_iota(jnp.int32, (1, c), 1) % V_J
    y = xb
    for b in _BITS:
        y = jnp.where((s_lane & b) != 0, pltpu.roll(y, (V_J - b) * tm, axis=0), y)
    xs = (y * m_ref[...]).astype(jnp.bfloat16)
    z = jnp.dot(xs, w_ref[...], preferred_element_type=jnp.float32)
    for b in _BITS:
        z = jnp.where((s_lane & b) != 0, pltpu.roll(z, b * tm, axis=0), z)
    g = jnp.maximum(z * s1_ref[...] + b1_ref[...] + xb, 0.0)
    u = jnp.dot(g.astype(jnp.bfloat16), wt_ref[...],
                preferred_element_type=jnp.float32)
    u = jnp.maximum(u + bt_ref[...], 0.0)
    out = jnp.maximum(u * sb_ref[...] + bb_ref[...] + xb, 0.0)
    for v in range(V_J):
        o_ref[v] = out[v * tm:(v + 1) * tm]


def _pick_tm(nt):
    tm = None
    d = 8
    while d <= min(nt, 256):
        if nt % d == 0:
            tm = d
        d += 8
    return nt if tm is None else tm


def kernel(x, linear_w, linear_b, feature_mask, bn1_scale, bn1_bias,
           bn_a_scale, bn_a_bias, bn_b_scale, bn_b_bias, wt, conv_b,
           idx_in, idx_out):
    n, c, t, v = x.shape
    assert v == V_J
    co = linear_w.shape[1]
    nt = n * t
    tm = _pick_tm(nt)

    mask = jnp.tanh(feature_mask) + 1.0
    b1 = bn1_bias + linear_b * bn1_scale
    wt_f = bn_a_scale.reshape(-1, 1) * wt
    bt_f = bn_a_bias @ wt + conv_b

    mask_r = jnp.repeat(mask, tm, axis=0)
    s1_r = jnp.repeat(bn1_scale, tm, axis=0)
    b1_r = jnp.repeat(b1, tm, axis=0)

    xt = jnp.transpose(x, (3, 0, 2, 1)).reshape(v, nt, c)

    out = pl.pallas_call(
        partial(_fused_kernel, tm=tm),
        out_shape=jax.ShapeDtypeStruct((v, nt, co), jnp.float32),
        grid=(nt // tm,),
        in_specs=[
            pl.BlockSpec((v, tm, c), lambda i: (0, i, 0)),
            pl.BlockSpec((v * tm, c), lambda i: (0, 0)),
            pl.BlockSpec((v * tm, co), lambda i: (0, 0)),
            pl.BlockSpec((v * tm, co), lambda i: (0, 0)),
            pl.BlockSpec((c, co), lambda i: (0, 0)),
            pl.BlockSpec((co, co), lambda i: (0, 0)),
            pl.BlockSpec((1, co), lambda i: (0, 0)),
            pl.BlockSpec((1, co), lambda i: (0, 0)),
            pl.BlockSpec((1, co), lambda i: (0, 0)),
        ],
        out_specs=pl.BlockSpec((v, tm, co), lambda i: (0, i, 0)),
        compiler_params=pltpu.CompilerParams(
            dimension_semantics=("parallel",),
            vmem_limit_bytes=_VMEM_LIMIT),
    )(xt, mask_r, s1_r, b1_r,
      linear_w.astype(jnp.bfloat16), wt_f.astype(jnp.bfloat16),
      bt_f, bn_b_scale, bn_b_bias)

    return jnp.transpose(out.reshape(v, n, t, co), (1, 3, 2, 0))


# speedup vs baseline: 6.3774x; 1.1690x over previous
from functools import partial

import jax
import jax.numpy as jnp
from jax.experimental import pallas as pl
from jax.experimental.pallas import tpu as pltpu

V_J = 25
_BITS = (1, 2, 4, 8, 16)
_VMEM_LIMIT = 64 * 1024 * 1024


def _fused_kernel(x_ref, m_ref, s1_ref, b1_ref, w_ref, wt_ref, bt_ref,
                  sb_ref, bb_ref, o_ref, *, tm):
    c = x_ref.shape[-1]
    rows = V_J * tm
    xb = x_ref[...]
    xbf = xb.astype(jnp.bfloat16)
    s_lane = jax.lax.broadcasted_iota(jnp.int32, (1, 1, c), 2) % V_J
    y = xbf
    for b in _BITS:
        y = jnp.where((s_lane & b) != 0,
                      jnp.concatenate([y[b:], y[:b]], axis=0), y)
    xs = y * m_ref[...]
    z = jnp.dot(xs.reshape(rows, c), w_ref[...],
                preferred_element_type=jnp.float32)
    t = (z.reshape(V_J, tm, c) * s1_ref[...] + b1_ref[...]).astype(jnp.bfloat16)
    for b in _BITS:
        t = jnp.where((s_lane & b) != 0,
                      jnp.concatenate([t[V_J - b:], t[:V_J - b]], axis=0), t)
    g = jnp.maximum(t + xbf, 0)
    u = jnp.dot(g.reshape(rows, c), wt_ref[...],
                preferred_element_type=jnp.float32)
    u = jnp.maximum(u + bt_ref[...], 0.0)
    out = jnp.maximum(u.reshape(V_J, tm, c) * sb_ref[...] + bb_ref[...] + xb,
                      0.0)
    o_ref[...] = out


def _pick_tm(nt):
    tm = None
    d = 8
    while d <= min(nt, 256):
        if nt % d == 0:
            tm = d
        d += 8
    return nt if tm is None else tm


def kernel(x, linear_w, linear_b, feature_mask, bn1_scale, bn1_bias,
           bn_a_scale, bn_a_bias, bn_b_scale, bn_b_bias, wt, conv_b,
           idx_in, idx_out):
    n, c, t, v = x.shape
    assert v == V_J
    co = linear_w.shape[1]
    nt = n * t
    tm = _pick_tm(nt)

    mask = jnp.tanh(feature_mask) + 1.0
    b1 = bn1_bias + linear_b * bn1_scale
    wt_f = bn_a_scale.reshape(-1, 1) * wt
    bt_f = bn_a_bias @ wt + conv_b

    vv = jnp.arange(V_J)[:, None]
    jj = jnp.arange(co)[None, :] % V_J
    src = (vv + jj) % V_J
    s1p = jnp.take_along_axis(bn1_scale, src, axis=0)
    b1p = jnp.take_along_axis(b1, src, axis=0)

    xt = jnp.transpose(x, (3, 0, 2, 1)).reshape(v, nt, c)

    out = pl.pallas_call(
        partial(_fused_kernel, tm=tm),
        out_shape=jax.ShapeDtypeStruct((v, nt, co), jnp.float32),
        grid=(nt // tm,),
        in_specs=[
            pl.BlockSpec((v, tm, c), lambda i: (0, i, 0)),
            pl.BlockSpec((v, 1, c), lambda i: (0, 0, 0)),
            pl.BlockSpec((v, 1, co), lambda i: (0, 0, 0)),
            pl.BlockSpec((v, 1, co), lambda i: (0, 0, 0)),
            pl.BlockSpec((c, co), lambda i: (0, 0)),
            pl.BlockSpec((co, co), lambda i: (0, 0)),
            pl.BlockSpec((1, co), lambda i: (0, 0)),
            pl.BlockSpec((1, 1, co), lambda i: (0, 0, 0)),
            pl.BlockSpec((1, 1, co), lambda i: (0, 0, 0)),
        ],
        out_specs=pl.BlockSpec((v, tm, co), lambda i: (0, i, 0)),
        compiler_params=pltpu.CompilerParams(
            dimension_semantics=("parallel",),
            vmem_limit_bytes=_VMEM_LIMIT),
    )(xt, mask.astype(jnp.bfloat16).reshape(v, 1, c),
      s1p.reshape(v, 1, co), b1p.reshape(v, 1, co),
      linear_w.astype(jnp.bfloat16), wt_f.astype(jnp.bfloat16),
      bt_f, bn_b_scale.reshape(1, 1, co), bn_b_bias.reshape(1, 1, co))

    return jnp.transpose(out.reshape(v, n, t, co), (1, 3, 2, 0))
